# Initial kernel scaffold; baseline (speedup 1.0000x reference)
import jax
import jax.numpy as jnp
from jax.experimental import pallas as pl


def kernel(point_cloud_pos, points_embeddings, points_color, points_dir, raydir, camrotc2w, campos, near, far):
    raise NotImplementedError("write your pallas kernel here")


# speedup vs baseline: 1.8221x; 1.8221x over previous
import jax
import jax.numpy as jnp
from jax.experimental import pallas as pl
from jax.experimental.pallas import tpu as pltpu
from jax.experimental.pallas import tpu_sc as plsc

_K = 8
_SR = 24
_RADIUS2 = 0.16
_QB = 128
_GW = 128
_TW = 128


def _knn_body(q_ref, qsq_ref, ptst_ref, ptsq_ref, idx_ref, d2_ref):
    q = q_ref[...]
    ptst = ptst_ref[...]
    qsq = qsq_ref[...]
    dot = jnp.dot(q, ptst, preferred_element_type=jnp.float32)
    d2 = qsq - 2.0 * dot + ptsq_ref[...]
    n = d2.shape[1]
    iota = jax.lax.broadcasted_iota(jnp.int32, d2.shape, 1)
    vals = d2
    for j in range(_K):
        m = jnp.min(vals, axis=1, keepdims=True)
        am = jnp.min(jnp.where(vals == m, iota, n), axis=1, keepdims=True)
        d2_ref[:, j:j + 1] = m
        idx_ref[:, j:j + 1] = am
        if j + 1 < _K:
            vals = jnp.where(iota == am, jnp.inf, vals)


def _knn(q, qsq, ptst, ptsq):
    nq = q.shape[0]
    n = ptst.shape[1]
    grid = nq // _QB
    return pl.pallas_call(
        _knn_body,
        grid=(grid,),
        in_specs=[
            pl.BlockSpec((_QB, 3), lambda i: (i, 0)),
            pl.BlockSpec((_QB, 1), lambda i: (i, 0)),
            pl.BlockSpec((3, n), lambda i: (0, 0)),
            pl.BlockSpec((1, n), lambda i: (0, 0)),
        ],
        out_specs=[
            pl.BlockSpec((_QB, _K), lambda i: (i, 0)),
            pl.BlockSpec((_QB, _K), lambda i: (i, 0)),
        ],
        out_shape=[
            jax.ShapeDtypeStruct((nq, _K), jnp.int32),
            jax.ShapeDtypeStruct((nq, _K), jnp.float32),
        ],
    )(q, qsq, ptst, ptsq)


def _sc_gather(table, idx):
    n_idx = idx.shape[0]
    idx2 = idx.reshape(1, n_idx)

    @pl.kernel(
        out_type=jax.ShapeDtypeStruct((n_idx, _TW), table.dtype),
        mesh=plsc.VectorSubcoreMesh(core_axis_name="core",
                                    subcore_axis_name="subcore"),
    )
    def k(x_hbm, i_hbm, o_hbm):
        def body(i_vmem, o_vmem):
            pltpu.sync_copy(x_hbm.at[i_vmem.at[0]], o_vmem)

        pltpu.emit_pipeline(
            body,
            grid=(n_idx // _GW,),
            in_specs=[pl.BlockSpec((1, _GW), index_map=lambda i: (0, i))],
            out_specs=[pl.BlockSpec((_GW, _TW), index_map=lambda i: (i, 0))],
            core_axis_name=("core", "subcore"),
            dimension_semantics=(pltpu.PARALLEL,),
        )(i_hbm, o_hbm)

    return k(table, idx2)


def kernel(point_cloud_pos, points_embeddings, points_color, points_dir,
           raydir, camrotc2w, campos, near, far):
    rd = raydir[0]
    r = rd.shape[0]
    t = jnp.linspace(near[0], far[0], _SR)
    raypos = campos[0][None, None, :] + rd[:, None, :] * t[None, :, None]
    q = raypos.reshape(-1, 3)
    nq = q.shape[0]
    n = point_cloud_pos.shape[0]

    qsq = jnp.sum(q * q, axis=-1, keepdims=True)
    ptsq = jnp.sum(point_cloud_pos * point_cloud_pos, axis=-1)[None, :]
    ptst = point_cloud_pos.T

    idx, knn_d2 = _knn(q, qsq, ptst, ptsq)

    sample_pidx = jnp.where(knn_d2 <= _RADIUS2, idx, -1)
    sample_pnt_mask = (sample_pidx >= 0).reshape(1, r, _SR, _K)
    pidx = jnp.maximum(sample_pidx, 0).reshape(-1)

    d = points_embeddings.shape[1]
    pad = _TW - (3 + d + 3 + 3)
    table = jnp.concatenate(
        [point_cloud_pos, points_embeddings, points_color, points_dir,
         jnp.zeros((n, pad), jnp.float32)], axis=1)

    g = _sc_gather(table, pidx)
    sampled_xyz = g[:, :3].reshape(1, r, _SR, _K, 3)
    sampled_embedding = g[:, 3:3 + d].reshape(1, r, _SR, _K, d)
    sampled_color = g[:, 3 + d:6 + d].reshape(1, r, _SR, _K, 3)
    sampled_dir = g[:, 6 + d:9 + d].reshape(1, r, _SR, _K, 3)

    sample_loc_cam_coor = ((raypos - campos[0][None, None, :]) @ camrotc2w[0])[None]
    sample_ray_dirs = jnp.broadcast_to(rd[:, None, :], (r, _SR, 3))[None]
    return (sampled_color, sampled_dir, sampled_embedding, sampled_xyz,
            sample_pnt_mask.reshape(1, r, _SR, _K), raypos[None],
            sample_loc_cam_coor, sample_ray_dirs)


# speedup vs baseline: 1.8225x; 1.0002x over previous
import jax
import jax.numpy as jnp
from jax.experimental import pallas as pl
from jax.experimental.pallas import tpu as pltpu
from jax.experimental.pallas import tpu_sc as plsc

_K = 8
_SR = 24
_RADIUS2 = 0.16
_QB = 128
_GW = 128
_TW = 128


def _knn_body(q_ref, qsq_ref, ptst_ref, ptsq_ref, idx_ref, d2_ref):
    q = q_ref[...]
    ptst = ptst_ref[...]
    qsq = qsq_ref[...]
    dot = jnp.dot(q, ptst, preferred_element_type=jnp.float32)
    d2 = qsq - 2.0 * dot + ptsq_ref[...]
    n = d2.shape[1]
    iota = jax.lax.broadcasted_iota(jnp.int32, d2.shape, 1)
    vals = d2
    for j in range(_K):
        m = jnp.min(vals, axis=1, keepdims=True)
        am = jnp.min(jnp.where(vals == m, iota, n), axis=1, keepdims=True)
        d2_ref[:, j:j + 1] = m
        idx_ref[:, j:j + 1] = am
        if j + 1 < _K:
            vals = jnp.where(iota == am, jnp.inf, vals)


def _knn(q, qsq, ptst, ptsq):
    nq = q.shape[0]
    n = ptst.shape[1]
    grid = nq // _QB
    return pl.pallas_call(
        _knn_body,
        grid=(grid,),
        in_specs=[
            pl.BlockSpec((_QB, 3), lambda i: (i, 0)),
            pl.BlockSpec((_QB, 1), lambda i: (i, 0)),
            pl.BlockSpec((3, n), lambda i: (0, 0)),
            pl.BlockSpec((1, n), lambda i: (0, 0)),
        ],
        out_specs=[
            pl.BlockSpec((_QB, _K), lambda i: (i, 0)),
            pl.BlockSpec((_QB, _K), lambda i: (i, 0)),
        ],
        out_shape=[
            jax.ShapeDtypeStruct((nq, _K), jnp.int32),
            jax.ShapeDtypeStruct((nq, _K), jnp.float32),
        ],
    )(q, qsq, ptst, ptsq)


def _sc_gather(table, idx):
    n_idx = idx.shape[0]
    idx2 = idx.reshape(1, n_idx)

    units = 32
    per_unit = n_idx // (_GW * units)

    @pl.kernel(
        out_type=jax.ShapeDtypeStruct((n_idx, _TW), table.dtype),
        mesh=plsc.VectorSubcoreMesh(core_axis_name="core",
                                    subcore_axis_name="subcore"),
    )
    def k(x_hbm, i_hbm, o_hbm):
        def body(i_vmem, o_vmem):
            pltpu.sync_copy(x_hbm.at[i_vmem.at[0]], o_vmem)

        pltpu.emit_pipeline(
            body,
            grid=(units, per_unit),
            in_specs=[pl.BlockSpec((1, _GW),
                                   index_map=lambda u, i: (0, u * per_unit + i))],
            out_specs=[pl.BlockSpec((_GW, _TW),
                                    index_map=lambda u, i: (u * per_unit + i, 0))],
            core_axis_name=("core", "subcore"),
            dimension_semantics=(pltpu.PARALLEL, pltpu.ARBITRARY),
        )(i_hbm, o_hbm)

    return k(table, idx2)


def kernel(point_cloud_pos, points_embeddings, points_color, points_dir,
           raydir, camrotc2w, campos, near, far):
    rd = raydir[0]
    r = rd.shape[0]
    t = jnp.linspace(near[0], far[0], _SR)
    raypos = campos[0][None, None, :] + rd[:, None, :] * t[None, :, None]
    q = raypos.reshape(-1, 3)
    nq = q.shape[0]
    n = point_cloud_pos.shape[0]

    qsq = jnp.sum(q * q, axis=-1, keepdims=True)
    ptsq = jnp.sum(point_cloud_pos * point_cloud_pos, axis=-1)[None, :]
    ptst = point_cloud_pos.T

    idx, knn_d2 = _knn(q, qsq, ptst, ptsq)

    sample_pidx = jnp.where(knn_d2 <= _RADIUS2, idx, -1)
    sample_pnt_mask = (sample_pidx >= 0).reshape(1, r, _SR, _K)
    pidx = jnp.maximum(sample_pidx, 0).reshape(-1)

    d = points_embeddings.shape[1]
    pad = _TW - (3 + d + 3 + 3)
    table = jnp.concatenate(
        [point_cloud_pos, points_embeddings, points_color, points_dir,
         jnp.zeros((n, pad), jnp.float32)], axis=1)

    g = _sc_gather(table, pidx)
    sampled_xyz = g[:, :3].reshape(1, r, _SR, _K, 3)
    sampled_embedding = g[:, 3:3 + d].reshape(1, r, _SR, _K, d)
    sampled_color = g[:, 3 + d:6 + d].reshape(1, r, _SR, _K, 3)
    sampled_dir = g[:, 6 + d:9 + d].reshape(1, r, _SR, _K, 3)

    sample_loc_cam_coor = ((raypos - campos[0][None, None, :]) @ camrotc2w[0])[None]
    sample_ray_dirs = jnp.broadcast_to(rd[:, None, :], (r, _SR, 3))[None]
    return (sampled_color, sampled_dir, sampled_embedding, sampled_xyz,
            sample_pnt_mask.reshape(1, r, _SR, _K), raypos[None],
            sample_loc_cam_coor, sample_ray_dirs)


# speedup vs baseline: 3.0304x; 1.6628x over previous
import functools

import jax
import jax.numpy as jnp
from jax import lax
from jax.experimental import pallas as pl
from jax.experimental.pallas import tpu as pltpu
from jax.experimental.pallas import tpu_sc as plsc

_K = 8
_SR = 24
_RADIUS2 = 0.16
_QB = 128
_GW = 128
_TW = 48
_NW = 32
_CH = 16


def _knn_body(q_ref, qsq_ref, ptst_ref, ptsq_ref, idx_ref, d2_ref):
    q = q_ref[...]
    ptst = ptst_ref[...]
    qsq = qsq_ref[...]
    dot = jnp.dot(q, ptst, preferred_element_type=jnp.float32)
    d2 = qsq - 2.0 * dot + ptsq_ref[...]
    n = d2.shape[1]
    iota = jax.lax.broadcasted_iota(jnp.int32, d2.shape, 1)
    vals = d2
    for j in range(_K):
        m = jnp.min(vals, axis=1, keepdims=True)
        am = jnp.min(jnp.where(vals == m, iota, n), axis=1, keepdims=True)
        d2_ref[:, j:j + 1] = m
        idx_ref[:, j:j + 1] = am
        if j + 1 < _K:
            vals = jnp.where(iota == am, jnp.inf, vals)


def _knn(q, qsq, ptst, ptsq):
    nq = q.shape[0]
    n = ptst.shape[1]
    grid = nq // _QB
    return pl.pallas_call(
        _knn_body,
        grid=(grid,),
        in_specs=[
            pl.BlockSpec((_QB, 3), lambda i: (i, 0)),
            pl.BlockSpec((_QB, 1), lambda i: (i, 0)),
            pl.BlockSpec((3, n), lambda i: (0, 0)),
            pl.BlockSpec((1, n), lambda i: (0, 0)),
        ],
        out_specs=[
            pl.BlockSpec((_QB, _K), lambda i: (i, 0)),
            pl.BlockSpec((_QB, _K), lambda i: (i, 0)),
        ],
        out_shape=[
            jax.ShapeDtypeStruct((nq, _K), jnp.int32),
            jax.ShapeDtypeStruct((nq, _K), jnp.float32),
        ],
    )(q, qsq, ptst, ptsq)


def _sc_gather(table, idx):
    n_idx = idx.shape[0]
    n_win = n_idx // _GW
    win_per_w = n_win // _NW
    n_ch = win_per_w // _CH
    idx2 = idx.reshape(n_win, _GW)

    @functools.partial(
        pl.kernel,
        out_type=jax.ShapeDtypeStruct((n_win, _GW, _TW), table.dtype),
        mesh=plsc.VectorSubcoreMesh(core_axis_name="c", subcore_axis_name="s"),
        compiler_params=pltpu.CompilerParams(use_tc_tiling_on_sc=False),
        scratch_types=[
            pltpu.VMEM((_CH, _GW), jnp.int32),
            pltpu.VMEM((_CH, _GW, _TW), jnp.float32),
            pltpu.SemaphoreType.DMA,
        ],
    )
    def k(table_hbm, idx_hbm, out_hbm, idx_v, rows_v, sem):
        wid = lax.axis_index("s") * 2 + lax.axis_index("c")
        base = wid * win_per_w

        @pl.loop(0, n_ch)
        def _(c):
            w0 = base + c * _CH
            pltpu.sync_copy(idx_hbm.at[pl.ds(w0, _CH)], idx_v)
            copies = [
                pltpu.async_copy(table_hbm.at[idx_v.at[j]], rows_v.at[j], sem)
                for j in range(_CH)
            ]
            for cp in copies:
                cp.wait()
            pltpu.sync_copy(rows_v, out_hbm.at[pl.ds(w0, _CH)])

    return k(table, idx2).reshape(n_idx, _TW)


def kernel(point_cloud_pos, points_embeddings, points_color, points_dir,
           raydir, camrotc2w, campos, near, far):
    rd = raydir[0]
    r = rd.shape[0]
    t = jnp.linspace(near[0], far[0], _SR)
    raypos = campos[0][None, None, :] + rd[:, None, :] * t[None, :, None]
    q = raypos.reshape(-1, 3)
    nq = q.shape[0]
    n = point_cloud_pos.shape[0]

    qsq = jnp.sum(q * q, axis=-1, keepdims=True)
    ptsq = jnp.sum(point_cloud_pos * point_cloud_pos, axis=-1)[None, :]
    ptst = point_cloud_pos.T

    idx, knn_d2 = _knn(q, qsq, ptst, ptsq)

    sample_pidx = jnp.where(knn_d2 <= _RADIUS2, idx, -1)
    sample_pnt_mask = (sample_pidx >= 0).reshape(1, r, _SR, _K)
    pidx = jnp.maximum(sample_pidx, 0).reshape(-1)

    d = points_embeddings.shape[1]
    pad = _TW - (3 + d + 3 + 3)
    table = jnp.concatenate(
        [point_cloud_pos, points_embeddings, points_color, points_dir,
         jnp.zeros((n, pad), jnp.float32)], axis=1).astype(jnp.float32)

    g = _sc_gather(table, pidx)
    sampled_xyz = g[:, :3].reshape(1, r, _SR, _K, 3)
    sampled_embedding = g[:, 3:3 + d].reshape(1, r, _SR, _K, d)
    sampled_color = g[:, 3 + d:6 + d].reshape(1, r, _SR, _K, 3)
    sampled_dir = g[:, 6 + d:9 + d].reshape(1, r, _SR, _K, 3)

    sample_loc_cam_coor = ((raypos - campos[0][None, None, :]) @ camrotc2w[0])[None]
    sample_ray_dirs = jnp.broadcast_to(rd[:, None, :], (r, _SR, 3))[None]
    return (sampled_color, sampled_dir, sampled_embedding, sampled_xyz,
            sample_pnt_mask.reshape(1, r, _SR, _K), raypos[None],
            sample_loc_cam_coor, sample_ray_dirs)
